# Initial kernel scaffold; baseline (speedup 1.0000x reference)
import jax
import jax.numpy as jnp
from jax.experimental import pallas as pl


def kernel(hidden, mask, direction, w):
    raise NotImplementedError("write your pallas kernel here")


# speedup vs baseline: 6.0305x; 6.0305x over previous
import functools

import jax
import jax.numpy as jnp
from jax import lax
from jax.experimental import pallas as pl
from jax.experimental.pallas import tpu as pltpu
from jax.experimental.pallas import tpu_sc as plsc

_N = 4096
_L = 16
_NB = _N // _L
_K = 20
_UNROLL = 8
_LN2 = 0.6931471805599453


def _vln(x):
    bits = plsc.bitcast(x, jnp.int32)
    e = jnp.right_shift(bits, 23) - 127
    mbits = jnp.bitwise_or(jnp.bitwise_and(bits, 0x7FFFFF), 0x3F800000)
    m = plsc.bitcast(mbits, jnp.float32)
    t = (m - 1.0) / (m + 1.0)
    t2 = t * t
    poly = 1.0 / 11.0
    poly = poly * t2 + 1.0 / 9.0
    poly = poly * t2 + 1.0 / 7.0
    poly = poly * t2 + 1.0 / 5.0
    poly = poly * t2 + 1.0 / 3.0
    poly = poly * t2 + 1.0
    return e.astype(jnp.float32) * _LN2 + 2.0 * t * poly


def _bf16r(x):
    bits = plsc.bitcast(x, jnp.int32)
    lsb = jnp.bitwise_and(jnp.right_shift(bits, 16), 1)
    rounded = jnp.bitwise_and(bits + 0x7FFF + lsb, jnp.int32(-0x10000))
    return plsc.bitcast(rounded, jnp.float32)


def _xlane(v, op):
    lanes = lax.iota(jnp.int32, _L)
    for s in (1, 2, 4, 8):
        perm = jnp.bitwise_xor(lanes, s)
        v = op(v, v.at[perm].get(mode="promise_in_bounds"))
    return v


_mesh = plsc.VectorSubcoreMesh(core_axis_name="c", subcore_axis_name="s")


@functools.partial(
    pl.kernel,
    out_type=jax.ShapeDtypeStruct((_L,), jnp.float32),
    mesh=_mesh,
    compiler_params=pltpu.CompilerParams(needs_layout_passes=False),
    scratch_types=[
        pltpu.VMEM((_N,), jnp.float32),
        pltpu.VMEM((_N,), jnp.float32),
        pltpu.VMEM((_N,), jnp.float32),
        pltpu.VMEM((_N,), jnp.float32),
        pltpu.VMEM((_N,), jnp.float32),
        pltpu.VMEM((_L,), jnp.float32),
    ],
)
def _sc_loss(mask_hbm, dir_hbm, w_hbm, out_hbm, m_v, d_v, w_v, a_v, p_v, o_v):
    cid = lax.axis_index("c")
    sid = lax.axis_index("s")

    @pl.when(jnp.logical_and(cid == 0, sid == 0))
    def _work():
        pltpu.sync_copy(mask_hbm, m_v)
        pltpu.sync_copy(dir_hbm, d_v)
        pltpu.sync_copy(w_hbm, w_v)
        lanes = lax.iota(jnp.int32, _L)

        def setup(bb, carry):
            for u in range(_UNROLL):
                o = (bb * _UNROLL + u) * _L
                d = d_v[pl.ds(o, _L)]
                a_v[pl.ds(o, _L)] = jnp.abs(d)
                p_v[pl.ds(o, _L)] = (_bf16r(m_v[pl.ds(o, _L)] * d)
                                     * _bf16r(w_v[pl.ds(o, _L)]))
            return carry

        lax.fori_loop(0, _NB // _UNROLL, setup, 0)

        def pick(k, acc):
            def scan_blocks(bb, carry):
                bv, bi = carry
                for u in range(_UNROLL):
                    o = (bb * _UNROLL + u) * _L
                    a = a_v[pl.ds(o, _L)]
                    pred = a > bv
                    bv = jnp.where(pred, a, bv)
                    bi = jnp.where(pred, o + lanes, bi)
                return bv, bi

            init = (jnp.full((_L,), -1.0, jnp.float32),
                    jnp.zeros((_L,), jnp.int32))
            bv, bi = lax.fori_loop(0, _NB // _UNROLL, scan_blocks, init)
            mx = _xlane(bv, jnp.maximum)
            cand = jnp.where(bv == mx, bi, jnp.int32(2**30))
            iv = _xlane(cand, jnp.minimum)
            g = plsc.load_gather(p_v, [iv])
            acc = acc + jnp.where(lanes == 0, g, 0.0)
            plsc.store_scatter(a_v, [iv],
                               jnp.full((_L,), -1.0, jnp.float32))
            return acc

        acc = lax.fori_loop(0, _K, pick, jnp.zeros((_L,), jnp.float32))
        xv = _xlane(acc, jnp.add)
        xv = jnp.abs(xv) + 1e-8
        o_v[...] = -_vln(xv)
        pltpu.sync_copy(o_v, out_hbm)


def kernel(hidden, mask, direction, w):
    del hidden
    out = _sc_loss(mask, direction, w)
    return out[0]


# speedup vs baseline: 7.0966x; 1.1768x over previous
import functools

import jax
import jax.numpy as jnp
from jax import lax
from jax.experimental import pallas as pl
from jax.experimental.pallas import tpu as pltpu
from jax.experimental.pallas import tpu_sc as plsc

_N = 4096
_L = 16
_NB = _N // _L
_K = 20
_UA = 8
_UC = 4
_LN2 = 0.6931471805599453


def _vln(x):
    bits = plsc.bitcast(x, jnp.int32)
    e = jnp.right_shift(bits, 23) - 127
    mbits = jnp.bitwise_or(jnp.bitwise_and(bits, 0x7FFFFF), 0x3F800000)
    m = plsc.bitcast(mbits, jnp.float32)
    t = (m - 1.0) / (m + 1.0)
    t2 = t * t
    poly = 1.0 / 11.0
    poly = poly * t2 + 1.0 / 9.0
    poly = poly * t2 + 1.0 / 7.0
    poly = poly * t2 + 1.0 / 5.0
    poly = poly * t2 + 1.0 / 3.0
    poly = poly * t2 + 1.0
    return e.astype(jnp.float32) * _LN2 + 2.0 * t * poly


def _bf16r(x):
    bits = plsc.bitcast(x, jnp.int32)
    lsb = jnp.bitwise_and(jnp.right_shift(bits, 16), 1)
    rounded = jnp.bitwise_and(bits + 0x7FFF + lsb, jnp.int32(-0x10000))
    return plsc.bitcast(rounded, jnp.float32)


_mesh = plsc.VectorSubcoreMesh(core_axis_name="c", subcore_axis_name="s")


@functools.partial(
    pl.kernel,
    out_type=jax.ShapeDtypeStruct((_L,), jnp.float32),
    mesh=_mesh,
    compiler_params=pltpu.CompilerParams(needs_layout_passes=False),
    scratch_types=[
        pltpu.VMEM((_N,), jnp.float32),
        pltpu.VMEM((_N,), jnp.float32),
        pltpu.VMEM((_N,), jnp.float32),
        pltpu.VMEM((_N + _L,), jnp.float32),
        pltpu.VMEM((_N + _L,), jnp.int32),
        pltpu.VMEM((_L,), jnp.float32),
        pltpu.SemaphoreType.DMA,
        pltpu.SemaphoreType.DMA,
        pltpu.SemaphoreType.DMA,
    ],
)
def _sc_loss(mask_hbm, dir_hbm, w_hbm, out_hbm,
             m_v, d_v, w_v, a_c, i_c, o_v, sem_d, sem_m, sem_w):
    cid = lax.axis_index("c")
    sid = lax.axis_index("s")

    @pl.when(jnp.logical_and(cid == 0, sid == 0))
    def _work():
        cp_d = pltpu.async_copy(dir_hbm, d_v, sem_d)
        cp_m = pltpu.async_copy(mask_hbm, m_v, sem_m)
        cp_w = pltpu.async_copy(w_hbm, w_v, sem_w)
        lanes = lax.iota(jnp.int32, _L)
        cp_d.wait()

        def max_pass(bb, mx):
            for u in range(_UA):
                o = (bb * _UA + u) * _L
                mx = jnp.maximum(mx, jnp.abs(d_v[pl.ds(o, _L)]))
            return mx

        mlane = lax.fori_loop(0, _NB // _UA, max_pass,
                              jnp.zeros((_L,), jnp.float32))
        t0 = jnp.min(mlane)

        def compact(thr):
            def body(bb, off):
                for u in range(_UC):
                    o = (bb * _UC + u) * _L
                    a = jnp.abs(d_v[pl.ds(o, _L)])
                    pred = a > thr
                    plsc.store_compressed(a_c.at[pl.ds(off, _L)], a,
                                          mask=pred)
                    plsc.store_compressed(i_c.at[pl.ds(off, _L)], o + lanes,
                                          mask=pred)
                    off = off + plsc.all_reduce_population_count(pred)[0]
                return off
            return lax.fori_loop(0, _NB // _UC, body, jnp.int32(0))

        def need_more(c):
            return c[1] < _K

        def refine(c):
            thr, _ = c
            thr = jnp.where(thr > 1e-30, thr * 0.5, jnp.float32(-1.0))
            return thr, compact(thr)

        _, cnt = lax.while_loop(need_more, refine,
                                (t0 * 2.0, jnp.int32(0)))
        a_c[pl.ds(cnt, _L)] = jnp.full((_L,), -1.0, jnp.float32)
        nbc = (cnt + _L - 1) // _L
        cp_m.wait()
        cp_w.wait()

        def pick(_, acc):
            def scan(b, carry):
                bv, bp = carry
                a = a_c[pl.ds(b * _L, _L)]
                pred = a > bv
                bv = jnp.where(pred, a, bv)
                bp = jnp.where(pred, b * _L + lanes, bp)
                return bv, bp

            init = (jnp.full((_L,), -2.0, jnp.float32),
                    jnp.zeros((_L,), jnp.int32))
            bv, bp = lax.fori_loop(0, nbc, scan, init)
            mx = jnp.max(bv)
            pos = jnp.min(jnp.where(bv == mx, bp, jnp.int32(2**30)))
            pv = jnp.full((_L,), pos, jnp.int32)
            oi = plsc.load_gather(i_c, [pv])
            mg = plsc.load_gather(m_v, [oi])
            dg = plsc.load_gather(d_v, [oi])
            wg = plsc.load_gather(w_v, [oi])
            pr = _bf16r(mg * dg) * _bf16r(wg)
            plsc.store_scatter(a_c, [pv], jnp.full((_L,), -2.0, jnp.float32))
            return acc + pr[0]

        acc = lax.fori_loop(0, _K, pick, jnp.float32(0.0))
        xv = jnp.abs(jnp.full((_L,), acc, jnp.float32)) + 1e-8
        o_v[...] = -_vln(xv)
        pltpu.sync_copy(o_v, out_hbm)


def kernel(hidden, mask, direction, w):
    del hidden
    out = _sc_loss(mask, direction, w)
    return out[0]


# speedup vs baseline: 7.5898x; 1.0695x over previous
import functools

import jax
import jax.numpy as jnp
from jax import lax
from jax.experimental import pallas as pl
from jax.experimental.pallas import tpu as pltpu
from jax.experimental.pallas import tpu_sc as plsc

_N = 4096
_L = 16
_NB = _N // _L
_K = 20
_UA = 8
_UC = 4
_LN2 = 0.6931471805599453


def _vln(x):
    bits = plsc.bitcast(x, jnp.int32)
    e = jnp.right_shift(bits, 23) - 127
    mbits = jnp.bitwise_or(jnp.bitwise_and(bits, 0x7FFFFF), 0x3F800000)
    m = plsc.bitcast(mbits, jnp.float32)
    t = (m - 1.0) / (m + 1.0)
    t2 = t * t
    poly = 1.0 / 11.0
    poly = poly * t2 + 1.0 / 9.0
    poly = poly * t2 + 1.0 / 7.0
    poly = poly * t2 + 1.0 / 5.0
    poly = poly * t2 + 1.0 / 3.0
    poly = poly * t2 + 1.0
    return e.astype(jnp.float32) * _LN2 + 2.0 * t * poly


def _bf16r(x):
    bits = plsc.bitcast(x, jnp.int32)
    lsb = jnp.bitwise_and(jnp.right_shift(bits, 16), 1)
    rounded = jnp.bitwise_and(bits + 0x7FFF + lsb, jnp.int32(-0x10000))
    return plsc.bitcast(rounded, jnp.float32)


_mesh = plsc.VectorSubcoreMesh(core_axis_name="c", subcore_axis_name="s",
                               num_cores=1, num_subcores=1)


@functools.partial(
    pl.kernel,
    out_type=jax.ShapeDtypeStruct((_L,), jnp.float32),
    mesh=_mesh,
    compiler_params=pltpu.CompilerParams(needs_layout_passes=False),
    scratch_types=[
        pltpu.VMEM((_N,), jnp.float32),
        pltpu.VMEM((_N,), jnp.float32),
        pltpu.VMEM((_N,), jnp.float32),
        pltpu.VMEM((_N + _L,), jnp.float32),
        pltpu.VMEM((_N + _L,), jnp.int32),
        pltpu.VMEM((_L,), jnp.float32),
        pltpu.SemaphoreType.DMA,
        pltpu.SemaphoreType.DMA,
        pltpu.SemaphoreType.DMA,
    ],
)
def _sc_loss(mask_hbm, dir_hbm, w_hbm, out_hbm,
             m_v, d_v, w_v, a_c, i_c, o_v, sem_d, sem_m, sem_w):
    cid = lax.axis_index("c")
    sid = lax.axis_index("s")

    @pl.when(jnp.logical_and(cid == 0, sid == 0))
    def _work():
        cp_d = pltpu.async_copy(dir_hbm, d_v, sem_d)
        cp_m = pltpu.async_copy(mask_hbm, m_v, sem_m)
        cp_w = pltpu.async_copy(w_hbm, w_v, sem_w)
        lanes = lax.iota(jnp.int32, _L)
        cp_d.wait()

        def max_pass(bb, mx):
            for u in range(_UA):
                o = (bb * _UA + u) * _L
                mx = jnp.maximum(mx, jnp.abs(d_v[pl.ds(o, _L)]))
            return mx

        mlane = lax.fori_loop(0, _NB // _UA, max_pass,
                              jnp.zeros((_L,), jnp.float32))
        t0 = jnp.min(mlane)

        def compact(thr):
            def body(bb, off):
                for u in range(_UC):
                    o = (bb * _UC + u) * _L
                    a = jnp.abs(d_v[pl.ds(o, _L)])
                    pred = a > thr
                    plsc.store_compressed(a_c.at[pl.ds(off, _L)], a,
                                          mask=pred)
                    plsc.store_compressed(i_c.at[pl.ds(off, _L)], o + lanes,
                                          mask=pred)
                    off = off + plsc.all_reduce_population_count(pred)[0]
                return off
            return lax.fori_loop(0, _NB // _UC, body, jnp.int32(0))

        def need_more(c):
            return c[1] < _K

        def refine(c):
            thr, _ = c
            thr = jnp.where(thr > 1e-30, thr * 0.5, jnp.float32(-1.0))
            return thr, compact(thr)

        _, cnt = lax.while_loop(need_more, refine,
                                (t0 * 2.0, jnp.int32(0)))
        a_c[pl.ds(cnt, _L)] = jnp.full((_L,), -1.0, jnp.float32)
        nbc = (cnt + _L - 1) // _L
        cp_m.wait()
        cp_w.wait()

        def pick(_, acc):
            def scan(b, carry):
                bv, bp = carry
                a = a_c[pl.ds(b * _L, _L)]
                pred = a > bv
                bv = jnp.where(pred, a, bv)
                bp = jnp.where(pred, b * _L + lanes, bp)
                return bv, bp

            init = (jnp.full((_L,), -2.0, jnp.float32),
                    jnp.zeros((_L,), jnp.int32))
            bv, bp = lax.fori_loop(0, nbc, scan, init)
            mx = jnp.max(bv)
            pos = jnp.min(jnp.where(bv == mx, bp, jnp.int32(2**30)))
            pv = jnp.full((_L,), pos, jnp.int32)
            oi = plsc.load_gather(i_c, [pv])
            mg = plsc.load_gather(m_v, [oi])
            dg = plsc.load_gather(d_v, [oi])
            wg = plsc.load_gather(w_v, [oi])
            pr = _bf16r(mg * dg) * _bf16r(wg)
            plsc.store_scatter(a_c, [pv], jnp.full((_L,), -2.0, jnp.float32))
            return acc + pr[0]

        acc = lax.fori_loop(0, _K, pick, jnp.float32(0.0))
        xv = jnp.abs(jnp.full((_L,), acc, jnp.float32)) + 1e-8
        o_v[...] = -_vln(xv)
        pltpu.sync_copy(o_v, out_hbm)


def kernel(hidden, mask, direction, w):
    del hidden
    out = _sc_loss(mask, direction, w)
    return out[0]
